# initial kernel scaffold (unmeasured)
import jax
import jax.numpy as jnp
from jax import lax
from jax.experimental import pallas as pl
from jax.experimental.pallas import tpu as pltpu


def kernel(
    x,
):
    def body(*refs):
        pass

    out_shape = jax.ShapeDtypeStruct(..., jnp.float32)
    return pl.pallas_call(body, out_shape=out_shape)(...)


# baseline (device time: 18693 ns/iter reference)
import jax
import jax.numpy as jnp
from jax import lax
from jax.experimental import pallas as pl
from jax.experimental.pallas import tpu as pltpu

N_DEV = 4


def kernel(x):
    m, n = x.shape
    m_chunk = m // N_DEV

    def body(x_ref, out_ref, send_bufs, recv_bufs, red_ref, send_sems, recv_sems):
        my = lax.axis_index("i")
        left = (my - 1) % N_DEV
        right = (my + 1) % N_DEV

        barrier_sem = pltpu.get_barrier_semaphore()
        for nbr in (left, right):
            pl.semaphore_signal(
                barrier_sem, inc=1,
                device_id=(nbr,), device_id_type=pl.DeviceIdType.MESH,
            )
        pl.semaphore_wait(barrier_sem, 2)

        def chunk_rows(c):
            return pl.ds(c * m_chunk, m_chunk)

        for s in range(N_DEV - 1):
            c_send = (my - s) % N_DEV
            if s == 0:
                send_bufs[s] = x_ref[chunk_rows(c_send), :]
            else:
                send_bufs[s] = recv_bufs[s - 1] + x_ref[chunk_rows(c_send), :]
            rdma = pltpu.make_async_remote_copy(
                src_ref=send_bufs.at[s],
                dst_ref=recv_bufs.at[s],
                send_sem=send_sems.at[s],
                recv_sem=recv_sems.at[s],
                device_id=(right,),
                device_id_type=pl.DeviceIdType.MESH,
            )
            rdma.start()
            rdma.wait()

        c_mine = (my + 1) % N_DEV
        red_ref[...] = recv_bufs[N_DEV - 2] + x_ref[chunk_rows(c_mine), :]
        out_ref[chunk_rows(c_mine), :] = red_ref[...]

        for t in range(N_DEV - 1):
            h = (N_DEV - 1) + t
            src = red_ref if t == 0 else recv_bufs.at[h - 1]
            rdma = pltpu.make_async_remote_copy(
                src_ref=src,
                dst_ref=recv_bufs.at[h],
                send_sem=send_sems.at[h],
                recv_sem=recv_sems.at[h],
                device_id=(right,),
                device_id_type=pl.DeviceIdType.MESH,
            )
            rdma.start()
            rdma.wait()
            c_recv = (my - t) % N_DEV
            out_ref[chunk_rows(c_recv), :] = recv_bufs[h]

    n_hops = 2 * (N_DEV - 1)
    return pl.pallas_call(
        body,
        out_shape=jax.ShapeDtypeStruct((m, n), x.dtype),
        in_specs=[pl.BlockSpec(memory_space=pltpu.VMEM)],
        out_specs=pl.BlockSpec(memory_space=pltpu.VMEM),
        scratch_shapes=[
            pltpu.VMEM((N_DEV - 1, m_chunk, n), x.dtype),
            pltpu.VMEM((n_hops, m_chunk, n), x.dtype),
            pltpu.VMEM((m_chunk, n), x.dtype),
            pltpu.SemaphoreType.DMA((n_hops,)),
            pltpu.SemaphoreType.DMA((n_hops,)),
        ],
        compiler_params=pltpu.CompilerParams(collective_id=0),
    )(x)


# device time: 11562 ns/iter; 1.6168x vs baseline; 1.6168x over previous
import jax
import jax.numpy as jnp
from jax import lax
from jax.experimental import pallas as pl
from jax.experimental.pallas import tpu as pltpu

N_DEV = 4


def kernel(x):
    m, n = x.shape
    m_chunk = m // N_DEV

    def body(x_ref, out_ref, rs_recv, ag_recv, red_ref,
             rs_send_sems, rs_recv_sems, ag_send_sems, ag_recv_sems):
        my = lax.axis_index("i")

        barrier_sem = pltpu.get_barrier_semaphore()
        for k in range(1, N_DEV):
            pl.semaphore_signal(
                barrier_sem, inc=1,
                device_id=((my + k) % N_DEV,),
                device_id_type=pl.DeviceIdType.MESH,
            )
        pl.semaphore_wait(barrier_sem, N_DEV - 1)

        def chunk_rows(c):
            return pl.ds(c * m_chunk, m_chunk)

        rs = []
        for k in range(1, N_DEV):
            peer = (my + k) % N_DEV
            rdma = pltpu.make_async_remote_copy(
                src_ref=x_ref.at[chunk_rows(peer), :],
                dst_ref=rs_recv.at[k - 1],
                send_sem=rs_send_sems.at[k - 1],
                recv_sem=rs_recv_sems.at[k - 1],
                device_id=(peer,),
                device_id_type=pl.DeviceIdType.MESH,
            )
            rdma.start()
            rs.append(rdma)
        for rdma in rs:
            rdma.wait_recv()
        red_ref[...] = (
            x_ref[chunk_rows(my), :] + rs_recv[0] + rs_recv[1] + rs_recv[2]
        )
        out_ref[chunk_rows(my), :] = red_ref[...]

        ag = []
        for k in range(1, N_DEV):
            peer = (my + k) % N_DEV
            rdma = pltpu.make_async_remote_copy(
                src_ref=red_ref,
                dst_ref=ag_recv.at[k - 1],
                send_sem=ag_send_sems.at[k - 1],
                recv_sem=ag_recv_sems.at[k - 1],
                device_id=(peer,),
                device_id_type=pl.DeviceIdType.MESH,
            )
            rdma.start()
            ag.append(rdma)
        for k in range(1, N_DEV):
            ag[k - 1].wait_recv()
            out_ref[chunk_rows((my - k) % N_DEV), :] = ag_recv[k - 1]

        for rdma in rs:
            rdma.wait_send()
        for rdma in ag:
            rdma.wait_send()

    return pl.pallas_call(
        body,
        out_shape=jax.ShapeDtypeStruct((m, n), x.dtype),
        in_specs=[pl.BlockSpec(memory_space=pltpu.VMEM)],
        out_specs=pl.BlockSpec(memory_space=pltpu.VMEM),
        scratch_shapes=[
            pltpu.VMEM((N_DEV - 1, m_chunk, n), x.dtype),
            pltpu.VMEM((N_DEV - 1, m_chunk, n), x.dtype),
            pltpu.VMEM((m_chunk, n), x.dtype),
            pltpu.SemaphoreType.DMA((N_DEV - 1,)),
            pltpu.SemaphoreType.DMA((N_DEV - 1,)),
            pltpu.SemaphoreType.DMA((N_DEV - 1,)),
            pltpu.SemaphoreType.DMA((N_DEV - 1,)),
        ],
        compiler_params=pltpu.CompilerParams(collective_id=0),
    )(x)
